# Initial kernel scaffold; baseline (speedup 1.0000x reference)
import jax
import jax.numpy as jnp
from jax.experimental import pallas as pl


def kernel(x, edge_index, edge_attr, batch, A1, b1, root1, bias1, A2, b2, root2, bias2):
    raise NotImplementedError("write your pallas kernel here")


# speedup vs baseline: 1.3487x; 1.3487x over previous
import functools

import jax
import jax.numpy as jnp
from jax import lax
from jax.experimental import pallas as pl
from jax.experimental.pallas import tpu as pltpu
from jax.experimental.pallas import tpu_sc as plsc

N = 10000
E = 160000
NG = 10
F_IN = 128
F_MID = 64

NC = 2
NS = 16
NW = NC * NS
BB = 128
EP = 163840
GW = EP // (NW * BB)
PW = EP // NW
NP = 12800
STRIPE = NP // NS

BE = 256
BN = 400


def _sc_gather(d):
    mesh = plsc.VectorSubcoreMesh(core_axis_name="c", subcore_axis_name="s")

    @functools.partial(
        pl.kernel,
        mesh=mesh,
        out_type=jax.ShapeDtypeStruct((EP, d), jnp.float32),
        scratch_types=[
            pltpu.VMEM((GW, BB), jnp.int32),
            pltpu.VMEM((BB, d), jnp.float32),
            pltpu.SemaphoreType.DMA,
        ],
    )
    def k(table_hbm, idx_hbm, out_hbm, idx_v, rows_v, sem):
        wid = lax.axis_index("s") * NC + lax.axis_index("c")
        pltpu.sync_copy(idx_hbm.at[pl.ds(wid * GW, GW)], idx_v)

        def body(j, carry):
            pltpu.async_copy(table_hbm.at[idx_v.at[j]], rows_v, sem).wait()
            pltpu.sync_copy(rows_v, out_hbm.at[pl.ds(wid * PW + j * BB, BB)])
            return carry

        lax.fori_loop(0, GW, body, 0)

    return k


def _sc_scatter(with_deg):
    mesh = plsc.VectorSubcoreMesh(core_axis_name="c", subcore_axis_name="s")
    out_type = [jax.ShapeDtypeStruct((2 * NP, 64), jnp.float32)]
    scratch = [
        pltpu.VMEM((GW, BB), jnp.int32),
        pltpu.VMEM((BB, 64), jnp.float32),
        pltpu.VMEM_SHARED((NP, 64), jnp.float32),
        pltpu.SemaphoreType.DMA,
    ]
    if with_deg:
        out_type.append(jax.ShapeDtypeStruct((2 * NP, 16), jnp.float32))
        scratch += [
            pltpu.VMEM((BB, 16), jnp.float32),
            pltpu.VMEM_SHARED((NP, 16), jnp.float32),
        ]

    @functools.partial(
        pl.kernel, mesh=mesh, out_type=tuple(out_type), scratch_types=scratch)
    def k(*refs):
        if with_deg:
            (msgs_hbm, dst_hbm, z64, z16, ones_hbm, agg_out, deg_out,
             idx_v, msg_v, acc_sh, sem, ones_v, deg_sh) = refs
        else:
            (msgs_hbm, dst_hbm, z64, agg_out,
             idx_v, msg_v, acc_sh, sem) = refs
        cid = lax.axis_index("c")
        sid = lax.axis_index("s")
        wid = sid * NC + cid
        pltpu.sync_copy(z64, acc_sh.at[pl.ds(sid * STRIPE, STRIPE)])
        if with_deg:
            pltpu.sync_copy(z16, deg_sh.at[pl.ds(sid * STRIPE, STRIPE)])
            pltpu.sync_copy(ones_hbm, ones_v)
        plsc.subcore_barrier()
        pltpu.sync_copy(dst_hbm.at[pl.ds(wid * GW, GW)], idx_v)

        def body(j, carry):
            pltpu.async_copy(
                msgs_hbm.at[pl.ds(wid * PW + j * BB, BB)], msg_v, sem).wait()
            pltpu.sync_copy(msg_v, acc_sh.at[idx_v.at[j]], add=True)
            if with_deg:
                pltpu.sync_copy(ones_v, deg_sh.at[idx_v.at[j]], add=True)
            return carry

        lax.fori_loop(0, GW, body, 0)
        plsc.subcore_barrier()
        row = cid * NP + sid * STRIPE
        pltpu.sync_copy(acc_sh.at[pl.ds(sid * STRIPE, STRIPE)],
                        agg_out.at[pl.ds(row, STRIPE)])
        if with_deg:
            pltpu.sync_copy(deg_sh.at[pl.ds(sid * STRIPE, STRIPE)],
                            deg_out.at[pl.ds(row, STRIPE)])

    return k


def _tc_edge(din):
    def body(xs_ref, a_ref, A_ref, b_ref, out_ref):
        a = a_ref[...]
        acc = jnp.zeros((BE, 64), jnp.float32)
        for i in range(din):
            w = jnp.maximum(a * A_ref[i:i + 1, :] + b_ref[i:i + 1, :], 0.0)
            acc = acc + xs_ref[:, i:i + 1] * w
        out_ref[...] = acc

    return pl.pallas_call(
        body,
        grid=(EP // BE,),
        in_specs=[
            pl.BlockSpec((BE, F_IN), lambda i: (i, 0)),
            pl.BlockSpec((BE, 1), lambda i: (i, 0)),
            pl.BlockSpec((din, 64), lambda i: (0, 0)),
            pl.BlockSpec((din, 64), lambda i: (0, 0)),
        ],
        out_specs=pl.BlockSpec((BE, 64), lambda i: (i, 0)),
        out_shape=jax.ShapeDtypeStruct((EP, 64), jnp.float32),
    )


def _tc_combine(din, pad_out):
    dout = F_IN if pad_out else F_MID

    def body(x_ref, root_ref, a0_ref, a1_ref, d0_ref, d1_ref, bias_ref,
             out_ref):
        agg = a0_ref[...] + a1_ref[...]
        deg = d0_ref[:, 0:1] + d1_ref[:, 0:1]
        m = jnp.dot(x_ref[...], root_ref[...],
                    preferred_element_type=jnp.float32)
        h = jnp.maximum(m + agg / jnp.maximum(deg, 1.0) + bias_ref[...], 0.0)
        if pad_out:
            h = jnp.concatenate([h, jnp.zeros((BN, F_IN - F_MID),
                                              jnp.float32)], axis=1)
        out_ref[...] = h

    off = NP // BN
    return pl.pallas_call(
        body,
        grid=(N // BN,),
        in_specs=[
            pl.BlockSpec((BN, din), lambda i: (i, 0)),
            pl.BlockSpec((din, 64), lambda i: (0, 0)),
            pl.BlockSpec((BN, 64), lambda i: (i, 0)),
            pl.BlockSpec((BN, 64), lambda i: (i + off, 0)),
            pl.BlockSpec((BN, 16), lambda i: (i, 0)),
            pl.BlockSpec((BN, 16), lambda i: (i + off, 0)),
            pl.BlockSpec((1, 64), lambda i: (0, 0)),
        ],
        out_specs=pl.BlockSpec((BN, dout), lambda i: (i, 0)),
        out_shape=jax.ShapeDtypeStruct((N, dout), jnp.float32),
    )


def _tc_pool():
    def body(h_ref, b_ref, out_ref):
        oh = (b_ref[...] == lax.broadcasted_iota(jnp.int32, (1, 16), 1))
        oh = oh.astype(jnp.float32)
        s = lax.dot_general(oh, h_ref[...], (((0,), (0,)), ((), ())),
                            preferred_element_type=jnp.float32)
        ones = jnp.ones((N, 1), jnp.float32)
        c = lax.dot_general(oh, ones, (((0,), (0,)), ((), ())),
                            preferred_element_type=jnp.float32)
        out_ref[...] = s / jnp.maximum(c, 1.0)

    return pl.pallas_call(
        body,
        in_specs=[
            pl.BlockSpec((N, 64), lambda: (0, 0)),
            pl.BlockSpec((N, 1), lambda: (0, 0)),
        ],
        out_specs=pl.BlockSpec((16, 64), lambda: (0, 0)),
        out_shape=jax.ShapeDtypeStruct((16, 64), jnp.float32),
    )


def kernel(x, edge_index, edge_attr, batch, A1, b1, root1, bias1,
           A2, b2, root2, bias2):
    pad = EP - E
    src = jnp.concatenate([edge_index[0], jnp.zeros((pad,), jnp.int32)])
    dst = jnp.concatenate([edge_index[1], jnp.full((pad,), N, jnp.int32)])
    a = jnp.concatenate([edge_attr[:, 0], jnp.zeros((pad,), jnp.float32)])
    src2 = src.reshape(EP // BB, BB)
    dst2 = dst.reshape(EP // BB, BB)
    a2 = a.reshape(EP, 1)

    z64 = jnp.zeros((STRIPE, 64), jnp.float32)
    z16 = jnp.zeros((STRIPE, 16), jnp.float32)
    ones = jnp.ones((BB, 16), jnp.float32)

    A1m = A1.reshape(F_IN, F_MID)
    b1m = b1.reshape(F_IN, F_MID)
    A2m = A2.reshape(F_MID, F_MID)
    b2m = b2.reshape(F_MID, F_MID)
    bias1m = bias1.reshape(1, F_MID)
    bias2m = bias2.reshape(1, F_MID)
    batch2 = batch.reshape(N, 1)

    xs = _sc_gather(F_IN)(x, src2)
    msgs = _tc_edge(F_IN)(xs, a2, A1m, b1m)

    def _dbg_scatter(m, with_deg):
        agg_full = jax.ops.segment_sum(m, dst, num_segments=NP)
        agg_p = jnp.concatenate([agg_full, jnp.zeros_like(agg_full)], axis=0)
        if not with_deg:
            return (agg_p,)
        deg_full = jax.ops.segment_sum(jnp.ones((EP, 16), jnp.float32), dst,
                                       num_segments=NP)
        return agg_p, jnp.concatenate([deg_full, jnp.zeros_like(deg_full)], 0)

    agg, deg = _dbg_scatter(msgs, True)
    h = _tc_combine(F_IN, True)(x, root1, agg, agg, deg, deg, bias1m)

    hs = _sc_gather(F_IN)(h, src2)
    msgs2 = _tc_edge(F_MID)(hs, a2, A2m, b2m)
    (agg2,) = _dbg_scatter(msgs2, False)
    h2 = _tc_combine(F_MID, False)(h[:, :F_MID], root2, agg2, agg2, deg, deg,
                                   bias2m)

    pooled = _tc_pool()(h2, batch2)
    return pooled[:NG]
